# Initial kernel scaffold; baseline (speedup 1.0000x reference)
import jax
import jax.numpy as jnp
from jax.experimental import pallas as pl


def kernel(node_features, params, edge_index, map_entry_idx):
    raise NotImplementedError("write your pallas kernel here")


# speedup vs baseline: 1.0881x; 1.0881x over previous
import jax
import jax.numpy as jnp
from jax.experimental import pallas as pl


def _mm(x, W):
    n, din = x.shape
    dout = W.shape[1]
    blk = 2000

    def body(x_ref, w_ref, o_ref):
        o_ref[...] = jnp.dot(x_ref[...], w_ref[...],
                             preferred_element_type=jnp.float32)

    return pl.pallas_call(
        body,
        grid=(n // blk,),
        in_specs=[pl.BlockSpec((blk, din), lambda i: (i, 0)),
                  pl.BlockSpec((din, dout), lambda i: (0, 0))],
        out_specs=pl.BlockSpec((blk, dout), lambda i: (i, 0)),
        out_shape=jax.ShapeDtypeStruct((n, dout), jnp.float32),
    )(x, W)


def _round(h, src, dst, Wm, bm, Wd, bd, n):
    m = _mm(h, Wm) + bm
    agg = jax.ops.segment_max(m[src], dst, num_segments=n)
    agg = jnp.where(jnp.isfinite(agg), agg, 0.0)
    return _mm(agg, Wd) + bd


def kernel(node_features, params, edge_index, map_entry_idx):
    src = edge_index[0]
    dst = edge_index[1]
    n = node_features.shape[0]
    h = node_features
    saved = h
    for i in range(3):
        h = _round(h, src, dst, params[f"Wm{i}"], params[f"bm{i}"],
                   params[f"Wd{i}"], params[f"bd{i}"], n)
    h = jnp.concatenate([saved, h], axis=-1)
    h = _round(h, src, dst, params["Wm3"], params["bm3"],
               params["Wd3"], params["bd3"], n)
    saved = h
    for i in range(4, 7):
        h = _round(h, src, dst, params[f"Wm{i}"], params[f"bm{i}"],
                   params[f"Wd{i}"], params[f"bd{i}"], n)
    h = jnp.concatenate([saved, h], axis=-1)
    h = _round(h, src, dst, params["Wm7"], params["bm7"],
               params["Wd7"], params["bd7"], n)
    x = h[map_entry_idx]
    x = jax.nn.relu(x @ params["W1"] + params["b1"])
    x = x @ params["W2"] + params["b2"]
    return x


# speedup vs baseline: 2.7711x; 2.5468x over previous
import functools

import jax
import jax.numpy as jnp
from jax import lax
from jax.experimental import pallas as pl
from jax.experimental.pallas import tpu as pltpu
from jax.experimental.pallas import tpu_sc as plsc

N = 50000
E = 800000
NB = 128
BW = 392
NPB = 4
NPAD = NB * BW
CH = 128
EPAD = E + 2 * CH
D = 128
DC = 80
BLK = 1568

_mesh = plsc.VectorSubcoreMesh(core_axis_name="c", subcore_axis_name="s")


@functools.partial(
    pl.kernel,
    out_type=jax.ShapeDtypeStruct((NPAD, D), jnp.float32),
    mesh=_mesh,
    scratch_types=[
        pltpu.VMEM((BW + 8, D), jnp.float32),
        pltpu.VMEM((2, CH, D), jnp.float32),
        pltpu.VMEM((2, CH), jnp.int32),
        pltpu.VMEM((2, CH), jnp.int32),
        pltpu.VMEM((136, 16), jnp.int32),
        pltpu.VMEM((D,), jnp.float32),
        pltpu.SemaphoreType.DMA,
    ],
)
def _sc_segmax(table, srcs, dstl, starts, bias, out,
               acc, rows, idxb, dlb, starts_v, bias_v, sem):
    wid = lax.axis_index("s") * 2 + lax.axis_index("c")
    pltpu.sync_copy(starts, starts_v)
    pltpu.sync_copy(bias, bias_v)
    neg_inf = jnp.full((16,), -jnp.inf, dtype=jnp.float32)
    lane = lax.iota(jnp.int32, 16)

    def bucket_body(b, _):
        se = starts_v[b, :]
        s = se[0]
        e = se[1]
        nb_row = b * BW

        def init_body(r, _):
            for j in range(D // 16):
                acc[r, pl.ds(j * 16, 16)] = neg_inf
            return 0

        lax.fori_loop(0, BW, init_body, 0)

        s_al = (s // 8) * 8
        nchunks = (e - s_al + CH - 1) // CH

        def chunk_body(g, _):
            base = s_al + g * CH
            pltpu.sync_copy(srcs.at[pl.ds(base, CH)], idxb.at[0])
            pltpu.sync_copy(dstl.at[pl.ds(base, CH)], dlb.at[0])
            pltpu.async_copy(table.at[idxb.at[0]], rows.at[0], sem).wait()
            kstart = s - base
            kend = e - base

            def group_body(g16, _):
                off = pl.multiple_of(g16 * 16, 16)
                kv = lane + off
                vdl = dlb[0, pl.ds(off, 16)]
                valid = (kv >= kstart) & (kv < kend)
                dl_vec = jnp.where(valid, vdl, BW)
                for j in range(16):
                    dl = dl_vec[j]
                    kk = off + j
                    for jj in range(DC // 16):
                        sl = pl.ds(jj * 16, 16)
                        acc[dl, sl] = jnp.maximum(acc[dl, sl], rows[0, kk, sl])
                return 0

            lax.fori_loop(0, CH // 16, group_body, 0)
            return 0

        lax.fori_loop(0, nchunks, chunk_body, 0)

        def wb_body(r, _):
            for j in range(D // 16):
                sl = pl.ds(j * 16, 16)
                v = acc[r, sl]
                fin = jnp.abs(v) < jnp.inf
                acc[r, sl] = jnp.where(fin, v + bias_v[sl], 0.0)
            return 0

        lax.fori_loop(0, BW, wb_body, 0)
        pltpu.sync_copy(acc.at[pl.ds(0, BW)], out.at[pl.ds(nb_row, BW)])
        return 0

    lax.fori_loop(wid * NPB, wid * NPB + NPB, bucket_body, 0)


def _tc_call(body, n_out, *args):
    outs = [jax.ShapeDtypeStruct((NPAD, D), jnp.float32)] * n_out
    in_specs = []
    for a in args:
        if a.shape[0] == NPAD:
            in_specs.append(pl.BlockSpec((BLK, a.shape[1]), lambda i: (i, 0)))
        else:
            in_specs.append(pl.BlockSpec(a.shape, lambda i: (0, 0)))
    out_specs = pl.BlockSpec((BLK, D), lambda i: (i, 0))
    if n_out > 1:
        out_specs = [out_specs] * n_out
        outs = tuple(outs)
    else:
        outs = outs[0]
    return pl.pallas_call(
        body,
        grid=(NPAD // BLK,),
        in_specs=in_specs,
        out_specs=out_specs,
        out_shape=outs,
    )(*args)


def _dot(a, b):
    return jnp.dot(a, b, preferred_element_type=jnp.float32)


def _tc_xA(x, A):
    def body(x_ref, a_ref, o_ref):
        o_ref[...] = _dot(x_ref[...], a_ref[...])
    return _tc_call(body, 1, x, A)


def _tc_uDA(u, Dm, A):
    def body(u_ref, d_ref, a_ref, o_ref):
        o_ref[...] = _dot(_dot(u_ref[...], d_ref[...]), a_ref[...])
    return _tc_call(body, 1, u, Dm, A)


def _tc_concat(x0, u, Dm, Alo, Ahi, Blo, Bhi):
    def body(x_ref, u_ref, d_ref, alo, ahi, blo, bhi, olo, ohi):
        t = _dot(u_ref[...], d_ref[...])
        olo[...] = _dot(x_ref[...], alo[...]) + _dot(t, blo[...])
        ohi[...] = _dot(x_ref[...], ahi[...]) + _dot(t, bhi[...])
    return _tc_call(body, 2, x0, u, Dm, Alo, Ahi, Blo, Bhi)


def _tc_merge(ulo, uhi, Dlo, Dhi, bd_t, A):
    def body(ul, uh, dl, dh, b_ref, a_ref, oh_ref, op_ref):
        h = _dot(ul[...], dl[...]) + _dot(uh[...], dh[...]) + b_ref[0:1, :]
        oh_ref[...] = h
        op_ref[...] = _dot(h, a_ref[...])
    return _tc_call(body, 2, ulo, uhi, Dlo, Dhi, bd_t, A)


def _tc_final(x8, Wd, bd, W1, b1, W2, b2):
    def body(x_ref, wd, bdr, w1, b1r, w2, b2r, o_ref):
        h = _dot(x_ref[...], wd[...]) + bdr[0:1, :]
        t = jax.nn.relu(_dot(h, w1[...]) + b1r[0:1, :])
        o_ref[...] = _dot(t, w2[...]) + b2r[0:1, :]
    return pl.pallas_call(
        body,
        out_shape=jax.ShapeDtypeStruct((8, 256), jnp.float32),
    )(x8, Wd, bd, W1, b1, W2, b2)


def _pad(m, r, c):
    return jnp.pad(m, ((0, r - m.shape[0]), (0, c - m.shape[1])))


def _padv(v, c):
    return jnp.pad(v, (0, c - v.shape[0]))


def kernel(node_features, params, edge_index, map_entry_idx):
    p = params
    src = edge_index[0]
    dst = edge_index[1]

    dst_s, src_s = lax.sort([dst, src], num_keys=1)
    bucket = dst_s // BW
    dstl = dst_s - bucket * BW
    bounds = jnp.searchsorted(dst_s, jnp.arange(NB + 1, dtype=jnp.int32) * BW,
                              method="scan_unrolled").astype(jnp.int32)
    starts = jnp.zeros((136, 16), jnp.int32)
    starts = starts.at[:NB, 0].set(bounds[:NB]).at[:NB, 1].set(bounds[1:])
    src_pad = _padv(src_s, EPAD)
    dstl_pad = _padv(dstl, EPAD)

    x0 = _pad(node_features, NPAD, D)

    def seg(table, bias):
        return _sc_segmax(table, src_pad, dstl_pad, starts, _padv(bias, D))

    Wm = {i: _pad(p[f"Wm{i}"], D, D) for i in (0, 1, 2, 4, 5, 6)}
    Wd = {i: _pad(p[f"Wd{i}"], D, D) for i in (0, 1, 2, 4, 5, 6)}

    u = seg(_tc_xA(x0, Wm[0]), p["bm0"])
    for i in (1, 2):
        u = seg(_tc_uDA(u, Wd[i - 1], Wm[i]),
                p[f"bd{i-1}"] @ p[f"Wm{i}"] + p[f"bm{i}"])
    plo, phi = _tc_concat(x0, u, Wd[2],
                          _pad(p["Wm3"][:70, :70], D, D),
                          _pad(p["Wm3"][:70, 70:], D, D),
                          _pad(p["Wm3"][70:, :70], D, D),
                          _pad(p["Wm3"][70:, 70:], D, D))
    v3 = p["bd2"] @ p["Wm3"][70:] + p["bm3"]
    ulo, uhi = seg(plo, v3[:70]), seg(phi, v3[70:])
    Dlo = _pad(p["Wd3"][:70], D, D)
    Dhi = _pad(p["Wd3"][70:], D, D)
    bd3_t = jnp.tile(_padv(p["bd3"], D)[None, :], (8, 1))
    h4, p4 = _tc_merge(ulo, uhi, Dlo, Dhi, bd3_t, Wm[4])
    u = seg(p4, p["bm4"])
    for i in (5, 6):
        u = seg(_tc_uDA(u, Wd[i - 1], Wm[i]),
                p[f"bd{i-1}"] @ p[f"Wm{i}"] + p[f"bm{i}"])
    plo, phi = _tc_concat(h4, u, Wd[6],
                          _pad(p["Wm7"][:70, :70], D, D),
                          _pad(p["Wm7"][:70, 70:], D, D),
                          _pad(p["Wm7"][70:, :70], D, D),
                          _pad(p["Wm7"][70:, 70:], D, D))
    v7 = p["bd6"] @ p["Wm7"][70:] + p["bm7"]
    ulo, uhi = seg(plo, v7[:70]), seg(phi, v7[70:])
    idx = jnp.asarray(map_entry_idx, jnp.int32)
    x8 = jnp.concatenate([lax.dynamic_slice(ulo, (idx, 0), (8, D)),
                          lax.dynamic_slice(uhi, (idx, 0), (8, D))], axis=1)
    Wd7 = jnp.zeros((2 * D, D), jnp.float32)
    Wd7 = Wd7.at[:70, :70].set(p["Wd7"][:70]).at[D:D + 70, :70].set(p["Wd7"][70:])
    bd7_t = jnp.tile(_padv(p["bd7"], D)[None, :], (8, 1))
    W1 = _pad(p["W1"], D, D)
    b1_t = jnp.tile(_padv(p["b1"], D)[None, :], (8, 1))
    W2 = _pad(p["W2"], D, 256)
    b2_t = jnp.tile(p["b2"][None, :], (8, 1))
    out8 = _tc_final(x8, Wd7, bd7_t, W1, b1_t, W2, b2_t)
    return out8[0]


# speedup vs baseline: 4.2263x; 1.5251x over previous
import functools

import jax
import jax.numpy as jnp
from jax import lax
from jax.experimental import pallas as pl
from jax.experimental.pallas import tpu as pltpu
from jax.experimental.pallas import tpu_sc as plsc

N = 50000
E = 800000
NB = 128
BW = 392
NPB = 4
NPAD = NB * BW
CH = 128
CAPE = 12288
EPAD = E + CAPE + CH
D = 128
DC = 80
BLK = 1568

_mesh = plsc.VectorSubcoreMesh(core_axis_name="c", subcore_axis_name="s")


@functools.partial(
    pl.kernel,
    out_type=jax.ShapeDtypeStruct((NPAD, D), jnp.float32),
    mesh=_mesh,
    scratch_types=[
        pltpu.VMEM((BW + 8, D), jnp.float32),
        pltpu.VMEM((2, CH, D), jnp.float32),
        pltpu.VMEM((CAPE,), jnp.int32),
        pltpu.VMEM((CAPE,), jnp.int32),
        pltpu.VMEM((136, 16), jnp.int32),
        pltpu.VMEM((D,), jnp.float32),
        pltpu.SemaphoreType.DMA,
        pltpu.SemaphoreType.DMA,
    ],
)
def _sc_segmax(table, srcs, dstl, starts, bias, out,
               acc, rows, idxl, dll, starts_v, bias_v, sem0, sem1):
    wid = lax.axis_index("s") * 2 + lax.axis_index("c")
    pltpu.sync_copy(starts, starts_v)
    pltpu.sync_copy(bias, bias_v)
    neg_inf = jnp.full((16,), -jnp.inf, dtype=jnp.float32)
    lane = lax.iota(jnp.int32, 16)
    sems = (sem0, sem1)

    def bucket_body(b, _):
        se = starts_v[b, :]
        s = se[0]
        e = se[1]

        def init_body(r, _):
            for j in range(D // 16):
                acc[r, pl.ds(j * 16, 16)] = neg_inf
            return 0

        lax.fori_loop(0, BW, init_body, 0)

        s_al = (s // 8) * 8
        nsup = (e - s_al + CAPE - 1) // CAPE

        def super_body(si, _):
            sbase = s_al + si * CAPE
            pltpu.sync_copy(srcs.at[pl.ds(sbase, CAPE)], idxl)
            pltpu.sync_copy(dstl.at[pl.ds(sbase, CAPE)], dll)
            nch = jnp.minimum((e - sbase + CH - 1) // CH, CAPE // CH)

            def gather(c, slot):
                return pltpu.make_async_copy(
                    table.at[idxl.at[pl.ds(c * CH, CH)]], rows.at[slot],
                    sems[slot])

            def compute(c, slot):
                kstart = s - (sbase + c * CH)
                kend = e - (sbase + c * CH)

                def group_body(g16, _):
                    off = pl.multiple_of(g16 * 16, 16)
                    kv = lane + off
                    vdl = dll[pl.ds(pl.multiple_of(c * CH, 16) + off, 16)]
                    valid = (kv >= kstart) & (kv < kend)
                    dl_vec = jnp.where(valid, vdl, BW)
                    for j in range(16):
                        dl = dl_vec[j]
                        kk = off + j
                        for jj in range(DC // 16):
                            sl = pl.ds(jj * 16, 16)
                            acc[dl, sl] = jnp.maximum(acc[dl, sl],
                                                      rows[slot, kk, sl])
                    return 0

                lax.fori_loop(0, CH // 16, group_body, 0)

            gather(0, 0).start()

            def pair_body(t, _):
                c0 = 2 * t

                @pl.when(c0 + 1 < nch)
                def _():
                    gather(c0 + 1, 1).start()

                gather(c0, 0).wait()
                compute(c0, 0)

                @pl.when(c0 + 2 < nch)
                def _():
                    gather(c0 + 2, 0).start()

                @pl.when(c0 + 1 < nch)
                def _():
                    gather(c0 + 1, 1).wait()
                    compute(c0 + 1, 1)

                return 0

            lax.fori_loop(0, (nch + 1) // 2, pair_body, 0)
            return 0

        lax.fori_loop(0, nsup, super_body, 0)

        def wb_body(r, _):
            for j in range(D // 16):
                sl = pl.ds(j * 16, 16)
                v = acc[r, sl]
                fin = jnp.abs(v) < jnp.inf
                acc[r, sl] = jnp.where(fin, v + bias_v[sl], 0.0)
            return 0

        lax.fori_loop(0, BW, wb_body, 0)
        pltpu.sync_copy(acc.at[pl.ds(0, BW)], out.at[pl.ds(b * BW, BW)])
        return 0

    lax.fori_loop(wid * NPB, wid * NPB + NPB, bucket_body, 0)


def _tc_call(body, n_out, *args):
    outs = [jax.ShapeDtypeStruct((NPAD, D), jnp.float32)] * n_out
    in_specs = []
    for a in args:
        if a.shape[0] == NPAD:
            in_specs.append(pl.BlockSpec((BLK, a.shape[1]), lambda i: (i, 0)))
        else:
            in_specs.append(pl.BlockSpec(a.shape, lambda i: (0, 0)))
    out_specs = pl.BlockSpec((BLK, D), lambda i: (i, 0))
    if n_out > 1:
        out_specs = [out_specs] * n_out
        outs = tuple(outs)
    else:
        outs = outs[0]
    return pl.pallas_call(
        body,
        grid=(NPAD // BLK,),
        in_specs=in_specs,
        out_specs=out_specs,
        out_shape=outs,
    )(*args)


def _dot(a, b):
    return jnp.dot(a, b, preferred_element_type=jnp.float32)


def _tc_xA(x, A):
    def body(x_ref, a_ref, o_ref):
        o_ref[...] = _dot(x_ref[...], a_ref[...])
    return _tc_call(body, 1, x, A)


def _tc_uDA(u, Dm, A):
    def body(u_ref, d_ref, a_ref, o_ref):
        o_ref[...] = _dot(_dot(u_ref[...], d_ref[...]), a_ref[...])
    return _tc_call(body, 1, u, Dm, A)


def _tc_concat(x0, u, Dm, Alo, Ahi, Blo, Bhi):
    def body(x_ref, u_ref, d_ref, alo, ahi, blo, bhi, olo, ohi):
        t = _dot(u_ref[...], d_ref[...])
        olo[...] = _dot(x_ref[...], alo[...]) + _dot(t, blo[...])
        ohi[...] = _dot(x_ref[...], ahi[...]) + _dot(t, bhi[...])
    return _tc_call(body, 2, x0, u, Dm, Alo, Ahi, Blo, Bhi)


def _tc_merge(ulo, uhi, Dlo, Dhi, bd_t, A):
    def body(ul, uh, dl, dh, b_ref, a_ref, oh_ref, op_ref):
        h = _dot(ul[...], dl[...]) + _dot(uh[...], dh[...]) + b_ref[0:1, :]
        oh_ref[...] = h
        op_ref[...] = _dot(h, a_ref[...])
    return _tc_call(body, 2, ulo, uhi, Dlo, Dhi, bd_t, A)


def _tc_final(x8, Wd, bd, W1, b1, W2, b2):
    def body(x_ref, wd, bdr, w1, b1r, w2, b2r, o_ref):
        h = _dot(x_ref[...], wd[...]) + bdr[0:1, :]
        t = jax.nn.relu(_dot(h, w1[...]) + b1r[0:1, :])
        o_ref[...] = _dot(t, w2[...]) + b2r[0:1, :]
    return pl.pallas_call(
        body,
        out_shape=jax.ShapeDtypeStruct((8, 256), jnp.float32),
    )(x8, Wd, bd, W1, b1, W2, b2)


def _pad(m, r, c):
    return jnp.pad(m, ((0, r - m.shape[0]), (0, c - m.shape[1])))


def _padv(v, c):
    return jnp.pad(v, (0, c - v.shape[0]))


def kernel(node_features, params, edge_index, map_entry_idx):
    p = params
    src = edge_index[0]
    dst = edge_index[1]

    dst_s, src_s = lax.sort([dst, src], num_keys=1)
    bucket = dst_s // BW
    dstl = dst_s - bucket * BW
    bounds = jnp.searchsorted(dst_s, jnp.arange(NB + 1, dtype=jnp.int32) * BW,
                              method="scan_unrolled").astype(jnp.int32)
    starts = jnp.zeros((136, 16), jnp.int32)
    starts = starts.at[:NB, 0].set(bounds[:NB]).at[:NB, 1].set(bounds[1:])
    src_pad = _padv(src_s, EPAD)
    dstl_pad = _padv(dstl, EPAD)

    x0 = _pad(node_features, NPAD, D)

    def seg(table, bias):
        return _sc_segmax(table, src_pad, dstl_pad, starts, _padv(bias, D))

    Wm = {i: _pad(p[f"Wm{i}"], D, D) for i in (0, 1, 2, 4, 5, 6)}
    Wd = {i: _pad(p[f"Wd{i}"], D, D) for i in (0, 1, 2, 4, 5, 6)}

    u = seg(_tc_xA(x0, Wm[0]), p["bm0"])
    for i in (1, 2):
        u = seg(_tc_uDA(u, Wd[i - 1], Wm[i]),
                p[f"bd{i-1}"] @ p[f"Wm{i}"] + p[f"bm{i}"])
    plo, phi = _tc_concat(x0, u, Wd[2],
                          _pad(p["Wm3"][:70, :70], D, D),
                          _pad(p["Wm3"][:70, 70:], D, D),
                          _pad(p["Wm3"][70:, :70], D, D),
                          _pad(p["Wm3"][70:, 70:], D, D))
    v3 = p["bd2"] @ p["Wm3"][70:] + p["bm3"]
    ulo, uhi = seg(plo, v3[:70]), seg(phi, v3[70:])
    Dlo = _pad(p["Wd3"][:70], D, D)
    Dhi = _pad(p["Wd3"][70:], D, D)
    bd3_t = jnp.tile(_padv(p["bd3"], D)[None, :], (8, 1))
    h4, p4 = _tc_merge(ulo, uhi, Dlo, Dhi, bd3_t, Wm[4])
    u = seg(p4, p["bm4"])
    for i in (5, 6):
        u = seg(_tc_uDA(u, Wd[i - 1], Wm[i]),
                p[f"bd{i-1}"] @ p[f"Wm{i}"] + p[f"bm{i}"])
    plo, phi = _tc_concat(h4, u, Wd[6],
                          _pad(p["Wm7"][:70, :70], D, D),
                          _pad(p["Wm7"][:70, 70:], D, D),
                          _pad(p["Wm7"][70:, :70], D, D),
                          _pad(p["Wm7"][70:, 70:], D, D))
    v7 = p["bd6"] @ p["Wm7"][70:] + p["bm7"]
    ulo, uhi = seg(plo, v7[:70]), seg(phi, v7[70:])
    idx = jnp.asarray(map_entry_idx, jnp.int32)
    x8 = jnp.concatenate([lax.dynamic_slice(ulo, (idx, 0), (8, D)),
                          lax.dynamic_slice(uhi, (idx, 0), (8, D))], axis=1)
    Wd7 = jnp.zeros((2 * D, D), jnp.float32)
    Wd7 = Wd7.at[:70, :70].set(p["Wd7"][:70]).at[D:D + 70, :70].set(p["Wd7"][70:])
    bd7_t = jnp.tile(_padv(p["bd7"], D)[None, :], (8, 1))
    W1 = _pad(p["W1"], D, D)
    b1_t = jnp.tile(_padv(p["b1"], D)[None, :], (8, 1))
    W2 = _pad(p["W2"], D, 256)
    b2_t = jnp.tile(p["b2"][None, :], (8, 1))
    out8 = _tc_final(x8, Wd7, bd7_t, W1, b1_t, W2, b2_t)
    return out8[0]


# speedup vs baseline: 4.3261x; 1.0236x over previous
import functools

import jax
import jax.numpy as jnp
from jax import lax
from jax.experimental import pallas as pl
from jax.experimental.pallas import tpu as pltpu
from jax.experimental.pallas import tpu_sc as plsc

N = 50000
E = 800000
NB = 128
BW = 392
NPB = 4
NPAD = NB * BW
CH = 128
CAPE = 4096
EPAD = E + CAPE + CH
D = 128
DC = 80
BLK = 1568

_mesh = plsc.VectorSubcoreMesh(core_axis_name="c", subcore_axis_name="s")


@functools.partial(
    pl.kernel,
    out_type=jax.ShapeDtypeStruct((NPAD, D), jnp.float32),
    mesh=_mesh,
    scratch_types=[
        [pltpu.VMEM(((BW + 8) * 16,), jnp.float32) for _ in range(DC // 16)],
        pltpu.VMEM((BW, D), jnp.float32),
        pltpu.VMEM((2, CH, D), jnp.float32),
        pltpu.VMEM((CAPE,), jnp.int32),
        pltpu.VMEM((CAPE,), jnp.int32),
        pltpu.VMEM((136 * 16,), jnp.int32),
        pltpu.VMEM((D,), jnp.float32),
        pltpu.SemaphoreType.DMA,
        pltpu.SemaphoreType.DMA,
    ],
)
def _sc_segmax(table, srcs, dstl, starts, bias, out,
               banks, stage, rows, idxl, dll, starts_v, bias_v, sem0, sem1):
    wid = lax.axis_index("s") * 2 + lax.axis_index("c")
    pltpu.sync_copy(starts, starts_v)
    pltpu.sync_copy(bias, bias_v)
    neg_inf = jnp.full((16,), -jnp.inf, dtype=jnp.float32)
    lane = lax.iota(jnp.int32, 16)
    sems = (sem0, sem1)

    def bucket_body(b, _):
        se = starts_v[pl.ds(pl.multiple_of(b * 16, 16), 16)]
        s = se[0]
        e = se[1]

        def init_body(r, _):
            ro = pl.multiple_of(r * 16, 16)
            for bank in banks:
                bank[pl.ds(ro, 16)] = neg_inf
            return 0

        lax.fori_loop(0, BW, init_body, 0)

        s_al = (s // 8) * 8
        nsup = (e - s_al + CAPE - 1) // CAPE

        def super_body(si, _):
            sbase = s_al + si * CAPE
            pltpu.sync_copy(srcs.at[pl.ds(sbase, CAPE)], idxl)
            pltpu.sync_copy(dstl.at[pl.ds(sbase, CAPE)], dll)
            nch = jnp.minimum((e - sbase + CH - 1) // CH, CAPE // CH)

            def gather(c, slot):
                return pltpu.make_async_copy(
                    table.at[idxl.at[pl.ds(c * CH, CH)]], rows.at[slot],
                    sems[slot])

            def compute(c, slot):
                kstart = s - (sbase + c * CH)
                kend = e - (sbase + c * CH)

                def group_body(g16, _):
                    off = pl.multiple_of(g16 * 16, 16)
                    kv = lane + off
                    vdl = dll[pl.ds(pl.multiple_of(c * CH, 16) + off, 16)]
                    valid = (kv >= kstart) & (kv < kend)
                    dl_vec = jnp.where(valid, vdl, BW)
                    for j in range(16):
                        dlo = pl.multiple_of(dl_vec[j] * 16, 16)
                        kk = off + j
                        for jj, bank in enumerate(banks):
                            sl = pl.ds(jj * 16, 16)
                            bank[pl.ds(dlo, 16)] = jnp.maximum(
                                bank[pl.ds(dlo, 16)], rows[slot, kk, sl])
                    return 0

                lax.fori_loop(0, CH // 16, group_body, 0)

            gather(0, 0).start()

            def pair_body(t, _):
                c0 = 2 * t

                @pl.when(c0 + 1 < nch)
                def _():
                    gather(c0 + 1, 1).start()

                gather(c0, 0).wait()
                compute(c0, 0)

                @pl.when(c0 + 2 < nch)
                def _():
                    gather(c0 + 2, 0).start()

                @pl.when(c0 + 1 < nch)
                def _():
                    gather(c0 + 1, 1).wait()
                    compute(c0 + 1, 1)

                return 0

            lax.fori_loop(0, (nch + 1) // 2, pair_body, 0)
            return 0

        lax.fori_loop(0, nsup, super_body, 0)

        zero16 = jnp.zeros((16,), jnp.float32)

        def wb_body(r, _):
            ro = pl.multiple_of(r * 16, 16)
            for j, bank in enumerate(banks):
                sl = pl.ds(j * 16, 16)
                v = bank[pl.ds(ro, 16)]
                fin = jnp.abs(v) < jnp.inf
                stage[r, sl] = jnp.where(fin, v + bias_v[sl], 0.0)
            for j in range(DC // 16, D // 16):
                stage[r, pl.ds(j * 16, 16)] = zero16
            return 0

        lax.fori_loop(0, BW, wb_body, 0)
        pltpu.sync_copy(stage, out.at[pl.ds(b * BW, BW)])
        return 0

    lax.fori_loop(wid * NPB, wid * NPB + NPB, bucket_body, 0)


def _tc_call(body, n_out, *args):
    outs = [jax.ShapeDtypeStruct((NPAD, D), jnp.float32)] * n_out
    in_specs = []
    for a in args:
        if a.shape[0] == NPAD:
            in_specs.append(pl.BlockSpec((BLK, a.shape[1]), lambda i: (i, 0)))
        else:
            in_specs.append(pl.BlockSpec(a.shape, lambda i: (0, 0)))
    out_specs = pl.BlockSpec((BLK, D), lambda i: (i, 0))
    if n_out > 1:
        out_specs = [out_specs] * n_out
        outs = tuple(outs)
    else:
        outs = outs[0]
    return pl.pallas_call(
        body,
        grid=(NPAD // BLK,),
        in_specs=in_specs,
        out_specs=out_specs,
        out_shape=outs,
    )(*args)


def _dot(a, b):
    return jnp.dot(a, b, preferred_element_type=jnp.float32)


def _tc_xA(x, A):
    def body(x_ref, a_ref, o_ref):
        o_ref[...] = _dot(x_ref[...], a_ref[...])
    return _tc_call(body, 1, x, A)


def _tc_uDA(u, Dm, A):
    def body(u_ref, d_ref, a_ref, o_ref):
        o_ref[...] = _dot(_dot(u_ref[...], d_ref[...]), a_ref[...])
    return _tc_call(body, 1, u, Dm, A)


def _tc_concat(x0, u, Dm, Alo, Ahi, Blo, Bhi):
    def body(x_ref, u_ref, d_ref, alo, ahi, blo, bhi, olo, ohi):
        t = _dot(u_ref[...], d_ref[...])
        olo[...] = _dot(x_ref[...], alo[...]) + _dot(t, blo[...])
        ohi[...] = _dot(x_ref[...], ahi[...]) + _dot(t, bhi[...])
    return _tc_call(body, 2, x0, u, Dm, Alo, Ahi, Blo, Bhi)


def _tc_merge(ulo, uhi, Dlo, Dhi, bd_t, A):
    def body(ul, uh, dl, dh, b_ref, a_ref, oh_ref, op_ref):
        h = _dot(ul[...], dl[...]) + _dot(uh[...], dh[...]) + b_ref[0:1, :]
        oh_ref[...] = h
        op_ref[...] = _dot(h, a_ref[...])
    return _tc_call(body, 2, ulo, uhi, Dlo, Dhi, bd_t, A)


def _tc_final(x8, Wd, bd, W1, b1, W2, b2):
    def body(x_ref, wd, bdr, w1, b1r, w2, b2r, o_ref):
        h = _dot(x_ref[...], wd[...]) + bdr[0:1, :]
        t = jax.nn.relu(_dot(h, w1[...]) + b1r[0:1, :])
        o_ref[...] = _dot(t, w2[...]) + b2r[0:1, :]
    return pl.pallas_call(
        body,
        out_shape=jax.ShapeDtypeStruct((8, 256), jnp.float32),
    )(x8, Wd, bd, W1, b1, W2, b2)


def _pad(m, r, c):
    return jnp.pad(m, ((0, r - m.shape[0]), (0, c - m.shape[1])))


def _padv(v, c):
    return jnp.pad(v, (0, c - v.shape[0]))


def kernel(node_features, params, edge_index, map_entry_idx):
    p = params
    src = edge_index[0]
    dst = edge_index[1]

    key = (dst.astype(jnp.uint32) << 16) | src.astype(jnp.uint32)
    key_s = lax.sort(key)
    dst_s = (key_s >> 16).astype(jnp.int32)
    src_s = (key_s & 0xFFFF).astype(jnp.int32)
    bucket = dst_s // BW
    dstl = dst_s - bucket * BW
    bounds = jnp.searchsorted(dst_s, jnp.arange(NB + 1, dtype=jnp.int32) * BW,
                              method="scan_unrolled").astype(jnp.int32)
    starts = jnp.zeros((136, 16), jnp.int32)
    starts = starts.at[:NB, 0].set(bounds[:NB]).at[:NB, 1].set(bounds[1:])
    starts = starts.reshape(136 * 16)
    src_pad = _padv(src_s, EPAD)
    dstl_pad = _padv(dstl, EPAD)

    x0 = _pad(node_features, NPAD, D)

    def seg(table, bias):
        return _sc_segmax(table, src_pad, dstl_pad, starts, _padv(bias, D))

    Wm = {i: _pad(p[f"Wm{i}"], D, D) for i in (0, 1, 2, 4, 5, 6)}
    Wd = {i: _pad(p[f"Wd{i}"], D, D) for i in (0, 1, 2, 4, 5, 6)}

    u = seg(_tc_xA(x0, Wm[0]), p["bm0"])
    for i in (1, 2):
        u = seg(_tc_uDA(u, Wd[i - 1], Wm[i]),
                p[f"bd{i-1}"] @ p[f"Wm{i}"] + p[f"bm{i}"])
    plo, phi = _tc_concat(x0, u, Wd[2],
                          _pad(p["Wm3"][:70, :70], D, D),
                          _pad(p["Wm3"][:70, 70:], D, D),
                          _pad(p["Wm3"][70:, :70], D, D),
                          _pad(p["Wm3"][70:, 70:], D, D))
    v3 = p["bd2"] @ p["Wm3"][70:] + p["bm3"]
    ulo, uhi = seg(plo, v3[:70]), seg(phi, v3[70:])
    Dlo = _pad(p["Wd3"][:70], D, D)
    Dhi = _pad(p["Wd3"][70:], D, D)
    bd3_t = jnp.tile(_padv(p["bd3"], D)[None, :], (8, 1))
    h4, p4 = _tc_merge(ulo, uhi, Dlo, Dhi, bd3_t, Wm[4])
    u = seg(p4, p["bm4"])
    for i in (5, 6):
        u = seg(_tc_uDA(u, Wd[i - 1], Wm[i]),
                p[f"bd{i-1}"] @ p[f"Wm{i}"] + p[f"bm{i}"])
    plo, phi = _tc_concat(h4, u, Wd[6],
                          _pad(p["Wm7"][:70, :70], D, D),
                          _pad(p["Wm7"][:70, 70:], D, D),
                          _pad(p["Wm7"][70:, :70], D, D),
                          _pad(p["Wm7"][70:, 70:], D, D))
    v7 = p["bd6"] @ p["Wm7"][70:] + p["bm7"]
    ulo, uhi = seg(plo, v7[:70]), seg(phi, v7[70:])
    idx = jnp.asarray(map_entry_idx, jnp.int32)
    x8 = jnp.concatenate([lax.dynamic_slice(ulo, (idx, 0), (8, D)),
                          lax.dynamic_slice(uhi, (idx, 0), (8, D))], axis=1)
    Wd7 = jnp.zeros((2 * D, D), jnp.float32)
    Wd7 = Wd7.at[:70, :70].set(p["Wd7"][:70]).at[D:D + 70, :70].set(p["Wd7"][70:])
    bd7_t = jnp.tile(_padv(p["bd7"], D)[None, :], (8, 1))
    W1 = _pad(p["W1"], D, D)
    b1_t = jnp.tile(_padv(p["b1"], D)[None, :], (8, 1))
    W2 = _pad(p["W2"], D, 256)
    b2_t = jnp.tile(p["b2"][None, :], (8, 1))
    out8 = _tc_final(x8, Wd7, bd7_t, W1, b1_t, W2, b2_t)
    return out8[0]
